# baseline (speedup 1.0000x reference)
import functools
import math

import jax
import jax.numpy as jnp
from jax import lax
from jax.experimental import pallas as pl
from jax.experimental.pallas import tpu as pltpu
from jax.experimental.pallas import tpu_sc as plsc

D_MODEL = 768
SCALE = math.sqrt(D_MODEL)

NUM_CORES = 2
NUM_SUBCORES = 16
NW = NUM_CORES * NUM_SUBCORES
ROWS = 4
COLS = 4096
BATCH = ROWS * COLS
BPW = BATCH // NW
WPR = COLS // BPW
CHUNK = 32
NCHUNK = BPW // CHUNK
NBUF = 4
NRING = NCHUNK // NBUF
LANES = 16

_mesh = plsc.VectorSubcoreMesh(core_axis_name="c", subcore_axis_name="s")


@functools.partial(
    pl.kernel,
    mesh=_mesh,
    out_type=jax.ShapeDtypeStruct((ROWS, COLS, D_MODEL), jnp.float32),
    scratch_types=[
        pltpu.VMEM((BPW,), jnp.int32),
        *[pltpu.VMEM((CHUNK, D_MODEL), jnp.float32) for _ in range(NBUF)],
        *[pltpu.SemaphoreType.DMA for _ in range(2 * NBUF)],
    ],
)
def _emb_lookup(idx_hbm, table_hbm, out_hbm, idx_v, *bufs_and_sems):
    bufs = bufs_and_sems[:NBUF]
    gsems = bufs_and_sems[NBUF:2 * NBUF]
    ssems = bufs_and_sems[2 * NBUF:]
    wid = lax.axis_index("s") * NUM_CORES + lax.axis_index("c")
    row = wid // WPR
    col0 = (wid % WPR) * BPW
    pltpu.sync_copy(idx_hbm.at[row, pl.ds(col0, BPW)], idx_v)

    def gather_desc(g, b):
        return pltpu.make_async_copy(
            table_hbm.at[idx_v.at[pl.ds(g * CHUNK, CHUNK)]],
            bufs[b], gsems[b])

    HALF = CHUNK // 2

    def store_half_desc(g, b, h):
        return pltpu.make_async_copy(
            bufs[b].at[pl.ds(h * HALF, HALF)],
            out_hbm.at[row, pl.ds(col0 + g * CHUNK + h * HALF, HALF)],
            ssems[b])

    def scale_half(buf, h):
        @plsc.parallel_loop(h * HALF, (h + 1) * HALF, step=1, unroll=2)
        def _row(r):
            for j in range(D_MODEL // LANES):
                sl = pl.ds(j * LANES, LANES)
                buf[r, sl] = buf[r, sl] * SCALE

    for g in range(NBUF):
        gather_desc(g, g).start()

    def ring(t, carry):
        for b in range(NBUF):
            g = t * NBUF + b
            pg = g + NBUF - 1
            pb = (b - 1) % NBUF

            @pl.when(jnp.logical_and(g >= 1, pg < NCHUNK))
            def _():
                store_half_desc(g - 1, pb, 0).wait()
                store_half_desc(g - 1, pb, 1).wait()
                gather_desc(pg, pb).start()

            gather_desc(g, b).wait()
            scale_half(bufs[b], 0)
            store_half_desc(g, b, 0).start()
            scale_half(bufs[b], 1)
            store_half_desc(g, b, 1).start()
        return carry

    lax.fori_loop(0, NRING, ring, 0)
    for g in range(NCHUNK - NBUF + 1, NCHUNK):
        store_half_desc(g, g % NBUF, 0).wait()
        store_half_desc(g, g % NBUF, 1).wait()


def kernel(x, table):
    return _emb_lookup(x.astype(jnp.int32), table)


# speedup vs baseline: 1.0659x; 1.0659x over previous
import functools
import math

import jax
import jax.numpy as jnp
from jax import lax
from jax.experimental import pallas as pl
from jax.experimental.pallas import tpu as pltpu
from jax.experimental.pallas import tpu_sc as plsc

D_MODEL = 768
SCALE = math.sqrt(D_MODEL)

NUM_CORES = 2
NUM_SUBCORES = 16
NW = NUM_CORES * NUM_SUBCORES
ROWS = 4
COLS = 4096
BATCH = ROWS * COLS
BPW = BATCH // NW
WPR = COLS // BPW
CHUNK = 32
NCHUNK = BPW // CHUNK
NBUF = 4
NRING = NCHUNK // NBUF
LANES = 16

_mesh = plsc.VectorSubcoreMesh(core_axis_name="c", subcore_axis_name="s")


@functools.partial(
    pl.kernel,
    mesh=_mesh,
    out_type=jax.ShapeDtypeStruct((ROWS, COLS, D_MODEL), jnp.float32),
    scratch_types=[
        pltpu.VMEM((BPW,), jnp.int32),
        *[pltpu.VMEM((CHUNK, D_MODEL), jnp.float32) for _ in range(NBUF)],
        *[pltpu.SemaphoreType.DMA for _ in range(2 * NBUF)],
    ],
)
def _emb_lookup(idx_hbm, table_hbm, out_hbm, idx_v, *bufs_and_sems):
    bufs = bufs_and_sems[:NBUF]
    gsems = bufs_and_sems[NBUF:2 * NBUF]
    ssems = bufs_and_sems[2 * NBUF:]
    wid = lax.axis_index("s") * NUM_CORES + lax.axis_index("c")
    row = wid // WPR
    col0 = (wid % WPR) * BPW
    pltpu.sync_copy(idx_hbm.at[row, pl.ds(col0, BPW)], idx_v)

    def gather_desc(g, b):
        return pltpu.make_async_copy(
            table_hbm.at[idx_v.at[pl.ds(g * CHUNK, CHUNK)]],
            bufs[b], gsems[b])

    def store_desc(g, b):
        return pltpu.make_async_copy(
            bufs[b], out_hbm.at[row, pl.ds(col0 + g * CHUNK, CHUNK)],
            ssems[b])

    def scale(buf):
        @plsc.parallel_loop(0, CHUNK, step=1, unroll=2)
        def _row(r):
            for j in range(D_MODEL // LANES):
                sl = pl.ds(j * LANES, LANES)
                buf[r, sl] = buf[r, sl] * SCALE

    for g in range(NBUF):
        gather_desc(g, g).start()

    def ring(t, carry):
        for b in range(NBUF):
            g = t * NBUF + b
            pg = g + NBUF - 1
            pb = (b - 1) % NBUF

            @pl.when(jnp.logical_and(g >= 1, pg < NCHUNK))
            def _():
                store_desc(g - 1, pb).wait()
                gather_desc(pg, pb).start()

            gather_desc(g, b).wait()
            scale(bufs[b])
            store_desc(g, b).start()
        return carry

    lax.fori_loop(0, NRING, ring, 0)
    for g in range(NCHUNK - NBUF + 1, NCHUNK):
        store_desc(g, g % NBUF).wait()


def kernel(x, table):
    return _emb_lookup(x.astype(jnp.int32), table)


# speedup vs baseline: 1.0869x; 1.0197x over previous
import functools
import math

import jax
import jax.numpy as jnp
from jax import lax
from jax.experimental import pallas as pl
from jax.experimental.pallas import tpu as pltpu
from jax.experimental.pallas import tpu_sc as plsc

D_MODEL = 768
SCALE = math.sqrt(D_MODEL)

NUM_CORES = 2
NUM_SUBCORES = 16
NW = NUM_CORES * NUM_SUBCORES
ROWS = 4
COLS = 4096
BATCH = ROWS * COLS
BPW = BATCH // NW
WPR = COLS // BPW
CHUNK = 32
NCHUNK = BPW // CHUNK
NBUF = 4
NRING = NCHUNK // NBUF
LANES = 16

_mesh = plsc.VectorSubcoreMesh(core_axis_name="c", subcore_axis_name="s")


@functools.partial(
    pl.kernel,
    mesh=_mesh,
    out_type=jax.ShapeDtypeStruct((ROWS, COLS, D_MODEL), jnp.float32),
    scratch_types=[
        pltpu.VMEM((BPW,), jnp.int32),
        *[pltpu.VMEM((CHUNK, D_MODEL), jnp.float32) for _ in range(NBUF)],
        *[pltpu.SemaphoreType.DMA for _ in range(2 * NBUF)],
    ],
)
def _emb_lookup(idx_hbm, table_hbm, out_hbm, idx_v, *bufs_and_sems):
    bufs = bufs_and_sems[:NBUF]
    gsems = bufs_and_sems[NBUF:2 * NBUF]
    ssems = bufs_and_sems[2 * NBUF:]
    wid = lax.axis_index("s") * NUM_CORES + lax.axis_index("c")
    row = wid // WPR
    col0 = (wid % WPR) * BPW
    pltpu.sync_copy(idx_hbm.at[row, pl.ds(col0, BPW)], idx_v)

    def gather_desc(g, b):
        return pltpu.make_async_copy(
            table_hbm.at[idx_v.at[pl.ds(g * CHUNK, CHUNK)]],
            bufs[b], gsems[b])

    def store_desc(g, b):
        return pltpu.make_async_copy(
            bufs[b], out_hbm.at[row, pl.ds(col0 + g * CHUNK, CHUNK)],
            ssems[b])

    def scale(buf):
        @plsc.parallel_loop(0, CHUNK, step=1, unroll=1)
        def _row(r):
            for j in range(D_MODEL // LANES):
                sl = pl.ds(j * LANES, LANES)
                buf[r, sl] = buf[r, sl] * SCALE

    for g in range(NBUF):
        gather_desc(g, g).start()

    def ring(t, carry):
        for b in range(NBUF):
            g = t * NBUF + b
            pg = g + NBUF - 1
            pb = (b - 1) % NBUF

            @pl.when(jnp.logical_and(g >= 1, pg < NCHUNK))
            def _():
                store_desc(g - 1, pb).wait()
                gather_desc(pg, pb).start()

            gather_desc(g, b).wait()
            scale(bufs[b])
            store_desc(g, b).start()
        return carry

    lax.fori_loop(0, NRING, ring, 0)
    for g in range(NCHUNK - NBUF + 1, NCHUNK):
        store_desc(g, g % NBUF).wait()


def kernel(x, table):
    return _emb_lookup(x.astype(jnp.int32), table)


# speedup vs baseline: 1.0967x; 1.0090x over previous
import functools
import math

import jax
import jax.numpy as jnp
from jax import lax
from jax.experimental import pallas as pl
from jax.experimental.pallas import tpu as pltpu
from jax.experimental.pallas import tpu_sc as plsc

D_MODEL = 768
SCALE = math.sqrt(D_MODEL)

NUM_CORES = 2
NUM_SUBCORES = 16
NW = NUM_CORES * NUM_SUBCORES
ROWS = 4
COLS = 4096
BATCH = ROWS * COLS
BPW = BATCH // NW
WPR = COLS // BPW
CHUNK = 32
NCHUNK = BPW // CHUNK
NBUF = 4
NRING = NCHUNK // NBUF
LANES = 16

_mesh = plsc.VectorSubcoreMesh(core_axis_name="c", subcore_axis_name="s")


@functools.partial(
    pl.kernel,
    mesh=_mesh,
    out_type=jax.ShapeDtypeStruct((ROWS, COLS, D_MODEL), jnp.float32),
    scratch_types=[
        pltpu.VMEM((BPW,), jnp.int32),
        *[pltpu.VMEM((CHUNK, D_MODEL), jnp.float32) for _ in range(NBUF)],
        *[pltpu.SemaphoreType.DMA for _ in range(2 * NBUF)],
    ],
)
def _emb_lookup(idx_hbm, table_hbm, out_hbm, idx_v, *bufs_and_sems):
    bufs = bufs_and_sems[:NBUF]
    gsems = bufs_and_sems[NBUF:2 * NBUF]
    ssems = bufs_and_sems[2 * NBUF:]
    wid = lax.axis_index("s") * NUM_CORES + lax.axis_index("c")
    row = wid // WPR
    col0 = (wid % WPR) * BPW
    pltpu.sync_copy(idx_hbm.at[row, pl.ds(col0, BPW)], idx_v)

    def gather_desc(g, b):
        return pltpu.make_async_copy(
            table_hbm.at[idx_v.at[pl.ds(g * CHUNK, CHUNK)]],
            bufs[b], gsems[b])

    def store_desc(g, b):
        return pltpu.make_async_copy(
            bufs[b], out_hbm.at[row, pl.ds(col0 + g * CHUNK, CHUNK)],
            ssems[b])

    def scale(buf):
        @plsc.parallel_loop(0, CHUNK, step=1, unroll=1)
        def _row(r):
            def inner(t, c):
                for u in range(8):
                    sl = pl.ds(t * (8 * LANES) + u * LANES, LANES)
                    buf[r, sl] = buf[r, sl] * SCALE
                return c
            lax.fori_loop(0, D_MODEL // (8 * LANES), inner, 0)

    for g in range(NBUF):
        gather_desc(g, g).start()

    def ring(t, carry):
        for b in range(NBUF):
            g = t * NBUF + b
            pg = g + NBUF - 1
            pb = (b - 1) % NBUF

            @pl.when(jnp.logical_and(g >= 1, pg < NCHUNK))
            def _():
                store_desc(g - 1, pb).wait()
                gather_desc(pg, pb).start()

            gather_desc(g, b).wait()
            scale(bufs[b])
            store_desc(g, b).start()
        return carry

    lax.fori_loop(0, NRING, ring, 0)
    for g in range(NCHUNK - NBUF + 1, NCHUNK):
        store_desc(g, g % NBUF).wait()


def kernel(x, table):
    return _emb_lookup(x.astype(jnp.int32), table)
